# Initial kernel scaffold; baseline (speedup 1.0000x reference)
import jax
import jax.numpy as jnp
from jax.experimental import pallas as pl


def kernel(x, edge_index, edge_weight, W1, b1, W2, b2):
    raise NotImplementedError("write your pallas kernel here")


# speedup vs baseline: 5.7188x; 5.7188x over previous
import functools

import jax
import jax.numpy as jnp
from jax import lax
from jax.experimental import pallas as pl
from jax.experimental.pallas import tpu as pltpu
from jax.experimental.pallas import tpu_sc as plsc

N = 10000
E = 320000
D = 128

NC = 2
NS = 16
NW = NC * NS

CHUNK = 80
NCHUNK = E // NW // CHUNK
SUB = 25
ACC_N = 10240
ROWS_PER_TILE = ACC_N // NS


def _mm_body(x_ref, w_ref, o_ref):
    o_ref[...] = jnp.dot(x_ref[...], w_ref[...],
                         preferred_element_type=jnp.float32)


def _tc_matmul(x, w, block=1000):
    grid = (x.shape[0] // block,)
    return pl.pallas_call(
        _mm_body,
        grid=grid,
        in_specs=[
            pl.BlockSpec((block, x.shape[1]), lambda i: (i, 0)),
            pl.BlockSpec((w.shape[0], w.shape[1]), lambda i: (0, 0)),
        ],
        out_specs=pl.BlockSpec((block, w.shape[1]), lambda i: (i, 0)),
        out_shape=jax.ShapeDtypeStruct((x.shape[0], w.shape[1]), jnp.float32),
    )(x, w)


def _fused_mm_body(p_ref, b_ref, w_ref, o_ref):
    h = jax.nn.relu(p_ref[0] + p_ref[1] + b_ref[...])
    o_ref[...] = jnp.dot(h, w_ref[...], preferred_element_type=jnp.float32)


def _tc_fused_matmul(parts, b, w, block=1000):
    grid = (N // block,)
    return pl.pallas_call(
        _fused_mm_body,
        grid=grid,
        in_specs=[
            pl.BlockSpec((2, block, D), lambda i: (0, i, 0)),
            pl.BlockSpec((1, D), lambda i: (0, 0)),
            pl.BlockSpec((D, D), lambda i: (0, 0)),
        ],
        out_specs=pl.BlockSpec((block, D), lambda i: (i, 0)),
        out_shape=jax.ShapeDtypeStruct((N, D), jnp.float32),
    )(parts, b.reshape(1, D), w)


def _bias_body(p_ref, b_ref, o_ref):
    o_ref[...] = p_ref[0] + p_ref[1] + b_ref[...]


def _tc_add_bias(parts, b, block=1000):
    grid = (N // block,)
    return pl.pallas_call(
        _bias_body,
        grid=grid,
        in_specs=[
            pl.BlockSpec((2, block, D), lambda i: (0, i, 0)),
            pl.BlockSpec((1, D), lambda i: (0, 0)),
        ],
        out_specs=pl.BlockSpec((block, D), lambda i: (i, 0)),
        out_shape=jax.ShapeDtypeStruct((N, D), jnp.float32),
    )(parts, b.reshape(1, D))


def _sc_agg_kernel(h_hbm, src_hbm, dst_hbm, w_hbm, out_hbm,
                   src_v, dst_v, w_v, rows_v, acc, sem):
    cid = lax.axis_index("c")
    sid = lax.axis_index("s")

    def zrow(i, _):
        for j in range(D // 16):
            rows_v[i, pl.ds(j * 16, 16)] = jnp.zeros((16,), jnp.float32)
        return 0
    lax.fori_loop(0, CHUNK, zrow, 0)
    base = sid * ROWS_PER_TILE
    off = 0
    while off < ROWS_PER_TILE:
        n = min(CHUNK, ROWS_PER_TILE - off)
        pltpu.sync_copy(rows_v.at[pl.ds(0, n)], acc.at[pl.ds(base + off, n)])
        off += n
    plsc.subcore_barrier()

    def super_body(s, _):
        pltpu.sync_copy(src_hbm.at[cid, sid, s], src_v)
        pltpu.sync_copy(dst_hbm.at[cid, sid, s], dst_v)
        pltpu.sync_copy(w_hbm.at[cid, sid, s], w_v)

        def chunk_body(t, _):
            pltpu.async_copy(h_hbm.at[src_v.at[t]], rows_v, sem).wait()

            def grp_body(g, _):
                wvec = w_v[t, pl.ds(g * 16, 16)]
                for l in range(16):
                    wl = wvec[l]
                    r = g * 16 + l
                    for j in range(D // 16):
                        sl = pl.ds(j * 16, 16)
                        rows_v[r, sl] = rows_v[r, sl] * wl
                return 0
            lax.fori_loop(0, CHUNK // 16, grp_body, 0)

            pltpu.sync_copy(rows_v, acc.at[dst_v.at[t]], add=True)
            return 0
        lax.fori_loop(0, SUB, chunk_body, 0)
        return 0
    lax.fori_loop(0, NCHUNK // SUB, super_body, 0)
    plsc.subcore_barrier()

    pltpu.sync_copy(acc.at[pl.ds(base, ROWS_PER_TILE)],
                    out_hbm.at[cid, pl.ds(base, ROWS_PER_TILE)])


def _sc_aggregate(h, src_r, dst_r, w_r):
    mesh = plsc.VectorSubcoreMesh(core_axis_name="c", subcore_axis_name="s",
                                  num_cores=NC, num_subcores=NS)
    return pl.kernel(
        _sc_agg_kernel,
        out_type=jax.ShapeDtypeStruct((NC, ACC_N, D), jnp.float32),
        mesh=mesh,
        scratch_types=[
            pltpu.VMEM((SUB, CHUNK), jnp.int32),
            pltpu.VMEM((SUB, CHUNK), jnp.int32),
            pltpu.VMEM((SUB, CHUNK), jnp.float32),
            pltpu.VMEM((CHUNK, D), jnp.float32),
            pltpu.VMEM_SHARED((ACC_N, D), jnp.float32),
            pltpu.SemaphoreType.DMA,
        ],
    )(h, src_r, dst_r, w_r)


def kernel(x, edge_index, edge_weight, W1, b1, W2, b2):
    eshape = (NC, NS, NCHUNK // SUB, SUB, CHUNK)
    src = edge_index[0].astype(jnp.int32).reshape(eshape)
    dst = edge_index[1].astype(jnp.int32).reshape(eshape)
    w = edge_weight.reshape(eshape)

    h1 = _tc_matmul(x, W1)
    p1 = _sc_aggregate(h1, src, dst, w)
    h2 = _tc_fused_matmul(p1, b1, W2)
    p2 = _sc_aggregate(h2, src, dst, w)
    return _tc_add_bias(p2, b2)


# speedup vs baseline: 8.4618x; 1.4796x over previous
import functools

import jax
import jax.numpy as jnp
from jax import lax
from jax.experimental import pallas as pl
from jax.experimental.pallas import tpu as pltpu
from jax.experimental.pallas import tpu_sc as plsc

N = 10000
E = 320000
D = 128

NC = 2
NS = 16
NW = NC * NS

CHUNK = 80
NCHUNK = E // NW // CHUNK
SUB = 25
ACC_N = 10240
ROWS_PER_TILE = ACC_N // NS


def _mm_body(x_ref, w_ref, o_ref):
    o_ref[...] = jnp.dot(x_ref[...], w_ref[...],
                         preferred_element_type=jnp.float32)


def _tc_matmul(x, w, block=1000):
    grid = (x.shape[0] // block,)
    return pl.pallas_call(
        _mm_body,
        grid=grid,
        in_specs=[
            pl.BlockSpec((block, x.shape[1]), lambda i: (i, 0)),
            pl.BlockSpec((w.shape[0], w.shape[1]), lambda i: (0, 0)),
        ],
        out_specs=pl.BlockSpec((block, w.shape[1]), lambda i: (i, 0)),
        out_shape=jax.ShapeDtypeStruct((x.shape[0], w.shape[1]), jnp.float32),
    )(x, w)


def _fused_mm_body(p_ref, b_ref, w_ref, o_ref):
    h = jax.nn.relu(p_ref[0] + p_ref[1] + b_ref[...])
    o_ref[...] = jnp.dot(h, w_ref[...], preferred_element_type=jnp.float32)


def _tc_fused_matmul(parts, b, w, block=1000):
    grid = (N // block,)
    return pl.pallas_call(
        _fused_mm_body,
        grid=grid,
        in_specs=[
            pl.BlockSpec((2, block, D), lambda i: (0, i, 0)),
            pl.BlockSpec((1, D), lambda i: (0, 0)),
            pl.BlockSpec((D, D), lambda i: (0, 0)),
        ],
        out_specs=pl.BlockSpec((block, D), lambda i: (i, 0)),
        out_shape=jax.ShapeDtypeStruct((N, D), jnp.float32),
    )(parts, b.reshape(1, D), w)


def _bias_body(p_ref, b_ref, o_ref):
    o_ref[...] = p_ref[0] + p_ref[1] + b_ref[...]


def _tc_add_bias(parts, b, block=1000):
    grid = (N // block,)
    return pl.pallas_call(
        _bias_body,
        grid=grid,
        in_specs=[
            pl.BlockSpec((2, block, D), lambda i: (0, i, 0)),
            pl.BlockSpec((1, D), lambda i: (0, 0)),
        ],
        out_specs=pl.BlockSpec((block, D), lambda i: (i, 0)),
        out_shape=jax.ShapeDtypeStruct((N, D), jnp.float32),
    )(parts, b.reshape(1, D))


def _scale_rows(rows_v, w_v, t):
    def grp_body(g, _):
        wvec = w_v[t, pl.ds(g * 16, 16)]
        for l in range(16):
            wl = wvec[l]
            r = g * 16 + l
            for j in range(D // 16):
                sl = pl.ds(j * 16, 16)
                rows_v[r, sl] = rows_v[r, sl] * wl
        return 0
    lax.fori_loop(0, CHUNK // 16, grp_body, 0)


def _sc_agg_kernel(h_hbm, src_hbm, dst_hbm, w_hbm, out_hbm,
                   src_v, dst_v, w_v, rows_a, rows_b, acc,
                   sem_g, sem_s):
    cid = lax.axis_index("c")
    sid = lax.axis_index("s")

    def zrow(i, _):
        for j in range(D // 16):
            rows_a[i, pl.ds(j * 16, 16)] = jnp.zeros((16,), jnp.float32)
        return 0
    lax.fori_loop(0, CHUNK, zrow, 0)
    base = sid * ROWS_PER_TILE
    off = 0
    while off < ROWS_PER_TILE:
        n = min(CHUNK, ROWS_PER_TILE - off)
        pltpu.sync_copy(rows_a.at[pl.ds(0, n)], acc.at[pl.ds(base + off, n)])
        off += n
    plsc.subcore_barrier()

    def gather(t, buf):
        return pltpu.make_async_copy(h_hbm.at[src_v.at[t]], buf, sem_g)

    def scatter(t, buf):
        return pltpu.async_copy(buf, acc.at[dst_v.at[t]], sem_s, add=True)

    def scatter_wait(t, buf):
        pltpu.make_async_copy(buf, acc.at[dst_v.at[t]], sem_s).wait()

    def super_body(s, _):
        pltpu.sync_copy(src_hbm.at[cid, sid, s], src_v)
        pltpu.sync_copy(dst_hbm.at[cid, sid, s], dst_v)
        pltpu.sync_copy(w_hbm.at[cid, sid, s], w_v)

        gather(0, rows_a).start()

        def chunk_body(t, _):
            def stage(cur, nxt):
                gather(t, cur).wait()

                @pl.when(t < SUB - 1)
                def _prefetch():
                    @pl.when(t >= 1)
                    def _free():
                        scatter_wait(t - 1, nxt)
                    gather(t + 1, nxt).start()

                _scale_rows(cur, w_v, t)
                scatter(t, cur)

            @pl.when(t % 2 == 0)
            def _even():
                stage(rows_a, rows_b)

            @pl.when(t % 2 == 1)
            def _odd():
                stage(rows_b, rows_a)
            return 0
        lax.fori_loop(0, SUB, chunk_body, 0)

        scatter_wait(SUB - 2, rows_b if (SUB - 2) % 2 else rows_a)
        scatter_wait(SUB - 1, rows_b if (SUB - 1) % 2 else rows_a)
        return 0
    lax.fori_loop(0, NCHUNK // SUB, super_body, 0)
    plsc.subcore_barrier()

    pltpu.sync_copy(acc.at[pl.ds(base, ROWS_PER_TILE)],
                    out_hbm.at[cid, pl.ds(base, ROWS_PER_TILE)])


def _sc_aggregate(h, src_r, dst_r, w_r):
    mesh = plsc.VectorSubcoreMesh(core_axis_name="c", subcore_axis_name="s",
                                  num_cores=NC, num_subcores=NS)
    return pl.kernel(
        _sc_agg_kernel,
        out_type=jax.ShapeDtypeStruct((NC, ACC_N, D), jnp.float32),
        mesh=mesh,
        scratch_types=[
            pltpu.VMEM((SUB, CHUNK), jnp.int32),
            pltpu.VMEM((SUB, CHUNK), jnp.int32),
            pltpu.VMEM((SUB, CHUNK), jnp.float32),
            pltpu.VMEM((CHUNK, D), jnp.float32),
            pltpu.VMEM((CHUNK, D), jnp.float32),
            pltpu.VMEM_SHARED((ACC_N, D), jnp.float32),
            pltpu.SemaphoreType.DMA,
            pltpu.SemaphoreType.DMA,
        ],
    )(h, src_r, dst_r, w_r)


def kernel(x, edge_index, edge_weight, W1, b1, W2, b2):
    eshape = (NC, NS, NCHUNK // SUB, SUB, CHUNK)
    src = edge_index[0].astype(jnp.int32).reshape(eshape)
    dst = edge_index[1].astype(jnp.int32).reshape(eshape)
    w = edge_weight.reshape(eshape)

    h1 = _tc_matmul(x, W1)
    p1 = _sc_aggregate(h1, src, dst, w)
    h2 = _tc_fused_matmul(p1, b1, W2)
    p2 = _sc_aggregate(h2, src, dst, w)
    return _tc_add_bias(p2, b2)


# speedup vs baseline: 8.4837x; 1.0026x over previous
import functools

import jax
import jax.numpy as jnp
from jax import lax
from jax.experimental import pallas as pl
from jax.experimental.pallas import tpu as pltpu
from jax.experimental.pallas import tpu_sc as plsc

N = 10000
E = 320000
D = 128

NC = 2
NS = 16
NW = NC * NS

CHUNK = 80
NCHUNK = E // NW // CHUNK
SUB = 25
ACC_N = 10240
ROWS_PER_TILE = ACC_N // NS


def _mm_body(x_ref, w_ref, o_ref):
    o_ref[...] = jnp.dot(x_ref[...], w_ref[...],
                         preferred_element_type=jnp.float32)


def _tc_matmul(x, w, block=1000):
    grid = (x.shape[0] // block,)
    return pl.pallas_call(
        _mm_body,
        grid=grid,
        in_specs=[
            pl.BlockSpec((block, x.shape[1]), lambda i: (i, 0)),
            pl.BlockSpec((w.shape[0], w.shape[1]), lambda i: (0, 0)),
        ],
        out_specs=pl.BlockSpec((block, w.shape[1]), lambda i: (i, 0)),
        out_shape=jax.ShapeDtypeStruct((x.shape[0], w.shape[1]), jnp.float32),
    )(x, w)


def _fused_mm_body(p_ref, b_ref, w_ref, o_ref):
    h = jax.nn.relu(p_ref[0] + p_ref[1] + b_ref[...])
    o_ref[...] = jnp.dot(h, w_ref[...], preferred_element_type=jnp.float32)


def _tc_fused_matmul(parts, b, w, block=1000):
    grid = (N // block,)
    return pl.pallas_call(
        _fused_mm_body,
        grid=grid,
        in_specs=[
            pl.BlockSpec((2, block, D), lambda i: (0, i, 0)),
            pl.BlockSpec((1, D), lambda i: (0, 0)),
            pl.BlockSpec((D, D), lambda i: (0, 0)),
        ],
        out_specs=pl.BlockSpec((block, D), lambda i: (i, 0)),
        out_shape=jax.ShapeDtypeStruct((N, D), jnp.float32),
    )(parts, b.reshape(1, D), w)


def _bias_body(p_ref, b_ref, o_ref):
    o_ref[...] = p_ref[0] + p_ref[1] + b_ref[...]


def _tc_add_bias(parts, b, block=1000):
    grid = (N // block,)
    return pl.pallas_call(
        _bias_body,
        grid=grid,
        in_specs=[
            pl.BlockSpec((2, block, D), lambda i: (0, i, 0)),
            pl.BlockSpec((1, D), lambda i: (0, 0)),
        ],
        out_specs=pl.BlockSpec((block, D), lambda i: (i, 0)),
        out_shape=jax.ShapeDtypeStruct((N, D), jnp.float32),
    )(parts, b.reshape(1, D))


def _scale_rows(rows_v, w_v, t):
    def grp_body(g, _):
        wvec = w_v[t, pl.ds(g * 16, 16)]
        for l in range(16):
            wl = wvec[l]
            r = g * 16 + l
            for j in range(D // 16):
                sl = pl.ds(j * 16, 16)
                rows_v[r, sl] = rows_v[r, sl] * wl
        return 0
    lax.fori_loop(0, CHUNK // 16, grp_body, 0)


def _sc_agg_kernel(h_hbm, src_hbm, dst_hbm, w_hbm, out_hbm,
                   src_v, dst_v, w_v, rows_a, rows_b, acc,
                   sem_g, sem_s):
    cid = lax.axis_index("c")
    sid = lax.axis_index("s")

    def zrow(i, _):
        for j in range(D // 16):
            rows_a[i, pl.ds(j * 16, 16)] = jnp.zeros((16,), jnp.float32)
        return 0
    lax.fori_loop(0, CHUNK, zrow, 0)
    base = sid * ROWS_PER_TILE
    off = 0
    while off < ROWS_PER_TILE:
        n = min(CHUNK, ROWS_PER_TILE - off)
        pltpu.sync_copy(rows_a.at[pl.ds(0, n)], acc.at[pl.ds(base + off, n)])
        off += n
    plsc.subcore_barrier()

    HALF = CHUNK // 2

    def gather_descs(t, buf):
        return (
            pltpu.make_async_copy(h_hbm.at[src_v.at[t, pl.ds(0, HALF)]],
                                  buf.at[pl.ds(0, HALF)], sem_g),
            pltpu.make_async_copy(h_hbm.at[src_v.at[t, pl.ds(HALF, HALF)]],
                                  buf.at[pl.ds(HALF, HALF)], sem_g),
        )

    def gather_start(t, buf):
        for d in gather_descs(t, buf):
            d.start()

    def gather_wait(t, buf):
        for d in gather_descs(t, buf):
            d.wait()

    def scatter(t, buf):
        return pltpu.async_copy(buf, acc.at[dst_v.at[t]], sem_s, add=True)

    def scatter_wait(t, buf):
        pltpu.make_async_copy(buf, acc.at[dst_v.at[t]], sem_s).wait()

    def super_body(s, _):
        pltpu.sync_copy(src_hbm.at[cid, sid, s], src_v)
        pltpu.sync_copy(dst_hbm.at[cid, sid, s], dst_v)
        pltpu.sync_copy(w_hbm.at[cid, sid, s], w_v)

        gather_start(0, rows_a)

        def chunk_body(t, _):
            def stage(cur, nxt):
                gather_wait(t, cur)

                @pl.when(t < SUB - 1)
                def _prefetch():
                    @pl.when(t >= 1)
                    def _free():
                        scatter_wait(t - 1, nxt)
                    gather_start(t + 1, nxt)

                _scale_rows(cur, w_v, t)
                scatter(t, cur)

            @pl.when(t % 2 == 0)
            def _even():
                stage(rows_a, rows_b)

            @pl.when(t % 2 == 1)
            def _odd():
                stage(rows_b, rows_a)
            return 0
        lax.fori_loop(0, SUB, chunk_body, 0)

        scatter_wait(SUB - 2, rows_b if (SUB - 2) % 2 else rows_a)
        scatter_wait(SUB - 1, rows_b if (SUB - 1) % 2 else rows_a)
        return 0
    lax.fori_loop(0, NCHUNK // SUB, super_body, 0)
    plsc.subcore_barrier()

    pltpu.sync_copy(acc.at[pl.ds(base, ROWS_PER_TILE)],
                    out_hbm.at[cid, pl.ds(base, ROWS_PER_TILE)])


def _sc_aggregate(h, src_r, dst_r, w_r):
    mesh = plsc.VectorSubcoreMesh(core_axis_name="c", subcore_axis_name="s",
                                  num_cores=NC, num_subcores=NS)
    return pl.kernel(
        _sc_agg_kernel,
        out_type=jax.ShapeDtypeStruct((NC, ACC_N, D), jnp.float32),
        mesh=mesh,
        scratch_types=[
            pltpu.VMEM((SUB, CHUNK), jnp.int32),
            pltpu.VMEM((SUB, CHUNK), jnp.int32),
            pltpu.VMEM((SUB, CHUNK), jnp.float32),
            pltpu.VMEM((CHUNK, D), jnp.float32),
            pltpu.VMEM((CHUNK, D), jnp.float32),
            pltpu.VMEM_SHARED((ACC_N, D), jnp.float32),
            pltpu.SemaphoreType.DMA,
            pltpu.SemaphoreType.DMA,
        ],
    )(h, src_r, dst_r, w_r)


def kernel(x, edge_index, edge_weight, W1, b1, W2, b2):
    eshape = (NC, NS, NCHUNK // SUB, SUB, CHUNK)
    src = edge_index[0].astype(jnp.int32).reshape(eshape)
    dst = edge_index[1].astype(jnp.int32).reshape(eshape)
    w = edge_weight.reshape(eshape)

    h1 = _tc_matmul(x, W1)
    p1 = _sc_aggregate(h1, src, dst, w)
    h2 = _tc_fused_matmul(p1, b1, W2)
    p2 = _sc_aggregate(h2, src, dst, w)
    return _tc_add_bias(p2, b2)


# speedup vs baseline: 8.6178x; 1.0158x over previous
import functools

import jax
import jax.numpy as jnp
from jax import lax
from jax.experimental import pallas as pl
from jax.experimental.pallas import tpu as pltpu
from jax.experimental.pallas import tpu_sc as plsc

N = 10000
E = 320000
D = 128

NC = 2
NS = 16
NW = NC * NS

CHUNK = 80
NCHUNK = E // NW // CHUNK
SUB = 25
ACC_N = 10240
ROWS_PER_TILE = ACC_N // NS


def _mm_body(x_ref, w_ref, o_ref):
    o_ref[...] = jnp.dot(x_ref[...], w_ref[...],
                         preferred_element_type=jnp.float32)


def _tc_matmul(x, w, block=2000):
    grid = (x.shape[0] // block,)
    return pl.pallas_call(
        _mm_body,
        grid=grid,
        in_specs=[
            pl.BlockSpec((block, x.shape[1]), lambda i: (i, 0)),
            pl.BlockSpec((w.shape[0], w.shape[1]), lambda i: (0, 0)),
        ],
        out_specs=pl.BlockSpec((block, w.shape[1]), lambda i: (i, 0)),
        out_shape=jax.ShapeDtypeStruct((x.shape[0], w.shape[1]), jnp.float32),
    )(x, w)


def _fused_mm_body(p_ref, b_ref, w_ref, o_ref):
    h = jax.nn.relu(p_ref[0] + p_ref[1] + b_ref[...])
    o_ref[...] = jnp.dot(h, w_ref[...], preferred_element_type=jnp.float32)


def _tc_fused_matmul(parts, b, w, block=2000):
    grid = (N // block,)
    return pl.pallas_call(
        _fused_mm_body,
        grid=grid,
        in_specs=[
            pl.BlockSpec((2, block, D), lambda i: (0, i, 0)),
            pl.BlockSpec((1, D), lambda i: (0, 0)),
            pl.BlockSpec((D, D), lambda i: (0, 0)),
        ],
        out_specs=pl.BlockSpec((block, D), lambda i: (i, 0)),
        out_shape=jax.ShapeDtypeStruct((N, D), jnp.float32),
    )(parts, b.reshape(1, D), w)


def _bias_body(p_ref, b_ref, o_ref):
    o_ref[...] = p_ref[0] + p_ref[1] + b_ref[...]


def _tc_add_bias(parts, b, block=2000):
    grid = (N // block,)
    return pl.pallas_call(
        _bias_body,
        grid=grid,
        in_specs=[
            pl.BlockSpec((2, block, D), lambda i: (0, i, 0)),
            pl.BlockSpec((1, D), lambda i: (0, 0)),
        ],
        out_specs=pl.BlockSpec((block, D), lambda i: (i, 0)),
        out_shape=jax.ShapeDtypeStruct((N, D), jnp.float32),
    )(parts, b.reshape(1, D))


def _scale_rows(rows_v, w_v, t):
    def grp_body(g, _):
        wvec = w_v[t, pl.ds(g * 16, 16)]
        for l in range(16):
            wl = wvec[l]
            r = g * 16 + l
            for j in range(D // 16):
                sl = pl.ds(j * 16, 16)
                rows_v[r, sl] = rows_v[r, sl] * wl
        return 0
    lax.fori_loop(0, CHUNK // 16, grp_body, 0)


def _sc_agg_kernel(h_hbm, src_hbm, dst_hbm, w_hbm, out_hbm,
                   src_v, dst_v, w_v, rows_a, rows_b, acc,
                   sem_g, sem_s):
    cid = lax.axis_index("c")
    sid = lax.axis_index("s")

    def zrow(i, _):
        for j in range(D // 16):
            rows_a[i, pl.ds(j * 16, 16)] = jnp.zeros((16,), jnp.float32)
        return 0
    lax.fori_loop(0, CHUNK, zrow, 0)
    base = sid * ROWS_PER_TILE
    off = 0
    while off < ROWS_PER_TILE:
        n = min(CHUNK, ROWS_PER_TILE - off)
        pltpu.sync_copy(rows_a.at[pl.ds(0, n)], acc.at[pl.ds(base + off, n)])
        off += n
    plsc.subcore_barrier()

    HALF = CHUNK // 2

    def gather_descs(t, buf):
        return (
            pltpu.make_async_copy(h_hbm.at[src_v.at[t, pl.ds(0, HALF)]],
                                  buf.at[pl.ds(0, HALF)], sem_g),
            pltpu.make_async_copy(h_hbm.at[src_v.at[t, pl.ds(HALF, HALF)]],
                                  buf.at[pl.ds(HALF, HALF)], sem_g),
        )

    def gather_start(t, buf):
        for d in gather_descs(t, buf):
            d.start()

    def gather_wait(t, buf):
        for d in gather_descs(t, buf):
            d.wait()

    def scatter(t, buf):
        return pltpu.async_copy(buf, acc.at[dst_v.at[t]], sem_s, add=True)

    def scatter_wait(t, buf):
        pltpu.make_async_copy(buf, acc.at[dst_v.at[t]], sem_s).wait()

    def super_body(s, _):
        pltpu.sync_copy(src_hbm.at[cid, sid, s], src_v)
        pltpu.sync_copy(dst_hbm.at[cid, sid, s], dst_v)
        pltpu.sync_copy(w_hbm.at[cid, sid, s], w_v)

        gather_start(0, rows_a)

        def chunk_body(t, _):
            def stage(cur, nxt):
                gather_wait(t, cur)

                @pl.when(t < SUB - 1)
                def _prefetch():
                    @pl.when(t >= 1)
                    def _free():
                        scatter_wait(t - 1, nxt)
                    gather_start(t + 1, nxt)

                _scale_rows(cur, w_v, t)
                scatter(t, cur)

            @pl.when(t % 2 == 0)
            def _even():
                stage(rows_a, rows_b)

            @pl.when(t % 2 == 1)
            def _odd():
                stage(rows_b, rows_a)
            return 0
        lax.fori_loop(0, SUB, chunk_body, 0)

        scatter_wait(SUB - 2, rows_b if (SUB - 2) % 2 else rows_a)
        scatter_wait(SUB - 1, rows_b if (SUB - 1) % 2 else rows_a)
        return 0
    lax.fori_loop(0, NCHUNK // SUB, super_body, 0)
    plsc.subcore_barrier()

    pltpu.sync_copy(acc.at[pl.ds(base, ROWS_PER_TILE)],
                    out_hbm.at[cid, pl.ds(base, ROWS_PER_TILE)])


def _sc_aggregate(h, src_r, dst_r, w_r):
    mesh = plsc.VectorSubcoreMesh(core_axis_name="c", subcore_axis_name="s",
                                  num_cores=NC, num_subcores=NS)
    return pl.kernel(
        _sc_agg_kernel,
        out_type=jax.ShapeDtypeStruct((NC, ACC_N, D), jnp.float32),
        mesh=mesh,
        scratch_types=[
            pltpu.VMEM((SUB, CHUNK), jnp.int32),
            pltpu.VMEM((SUB, CHUNK), jnp.int32),
            pltpu.VMEM((SUB, CHUNK), jnp.float32),
            pltpu.VMEM((CHUNK, D), jnp.float32),
            pltpu.VMEM((CHUNK, D), jnp.float32),
            pltpu.VMEM_SHARED((ACC_N, D), jnp.float32),
            pltpu.SemaphoreType.DMA,
            pltpu.SemaphoreType.DMA,
        ],
    )(h, src_r, dst_r, w_r)


def kernel(x, edge_index, edge_weight, W1, b1, W2, b2):
    eshape = (NC, NS, NCHUNK // SUB, SUB, CHUNK)
    src = edge_index[0].astype(jnp.int32).reshape(eshape)
    dst = edge_index[1].astype(jnp.int32).reshape(eshape)
    w = edge_weight.reshape(eshape)

    h1 = _tc_matmul(x, W1)
    p1 = _sc_aggregate(h1, src, dst, w)
    h2 = _tc_fused_matmul(p1, b1, W2)
    p2 = _sc_aggregate(h2, src, dst, w)
    return _tc_add_bias(p2, b2)


# speedup vs baseline: 8.8843x; 1.0309x over previous
import functools

import jax
import jax.numpy as jnp
from jax import lax
from jax.experimental import pallas as pl
from jax.experimental.pallas import tpu as pltpu
from jax.experimental.pallas import tpu_sc as plsc

N = 10000
E = 320000
D = 128

NC = 2
NS = 16
NW = NC * NS

CHUNK = 80
NCHUNK = E // NW // CHUNK
SUB = 25
ACC_N = 10240
ROWS_PER_TILE = ACC_N // NS


def _mm_body(x_ref, w_ref, o_ref):
    o_ref[...] = jnp.dot(x_ref[...], w_ref[...],
                         preferred_element_type=jnp.float32)


def _tc_matmul(x, w, block=2000):
    grid = (x.shape[0] // block,)
    return pl.pallas_call(
        _mm_body,
        grid=grid,
        in_specs=[
            pl.BlockSpec((block, x.shape[1]), lambda i: (i, 0)),
            pl.BlockSpec((w.shape[0], w.shape[1]), lambda i: (0, 0)),
        ],
        out_specs=pl.BlockSpec((block, w.shape[1]), lambda i: (i, 0)),
        out_shape=jax.ShapeDtypeStruct((x.shape[0], w.shape[1]), jnp.float32),
    )(x, w)


def _fused_mm_body(p_ref, b_ref, w_ref, o_ref):
    h = jax.nn.relu(p_ref[0] + p_ref[1] + b_ref[...])
    o_ref[...] = jnp.dot(h, w_ref[...], preferred_element_type=jnp.float32)


def _tc_fused_matmul(parts, b, w, block=2000):
    grid = (N // block,)
    return pl.pallas_call(
        _fused_mm_body,
        grid=grid,
        in_specs=[
            pl.BlockSpec((2, block, D), lambda i: (0, i, 0)),
            pl.BlockSpec((1, D), lambda i: (0, 0)),
            pl.BlockSpec((D, D), lambda i: (0, 0)),
        ],
        out_specs=pl.BlockSpec((block, D), lambda i: (i, 0)),
        out_shape=jax.ShapeDtypeStruct((N, D), jnp.float32),
    )(parts, b.reshape(1, D), w)


def _bias_body(p_ref, b_ref, o_ref):
    o_ref[...] = p_ref[0] + p_ref[1] + b_ref[...]


def _tc_add_bias(parts, b, block=2000):
    grid = (N // block,)
    return pl.pallas_call(
        _bias_body,
        grid=grid,
        in_specs=[
            pl.BlockSpec((2, block, D), lambda i: (0, i, 0)),
            pl.BlockSpec((1, D), lambda i: (0, 0)),
        ],
        out_specs=pl.BlockSpec((block, D), lambda i: (i, 0)),
        out_shape=jax.ShapeDtypeStruct((N, D), jnp.float32),
    )(parts, b.reshape(1, D))


def _scale_rows(rows_v, w_v, t):
    def grp_body(g, _):
        wvec = w_v[t, pl.ds(g * 16, 16)]
        for l in range(16):
            wl = wvec[l]
            r = g * 16 + l
            for j in range(D // 16):
                sl = pl.ds(j * 16, 16)
                rows_v[r, sl] = rows_v[r, sl] * wl
        return 0
    lax.fori_loop(0, CHUNK // 16, grp_body, 0)


def _sc_agg_kernel(h_hbm, src_hbm, dst_hbm, w_hbm, out_hbm,
                   src_v, dst_v, w_v, rows_a, rows_b, acc,
                   sem_g, sem_s):
    cid = lax.axis_index("c")
    sid = lax.axis_index("s")

    def zrow(i, _):
        for j in range(D // 16):
            rows_a[i, pl.ds(j * 16, 16)] = jnp.zeros((16,), jnp.float32)
        return 0
    lax.fori_loop(0, CHUNK, zrow, 0)
    base = sid * ROWS_PER_TILE
    zcopies = [
        pltpu.make_async_copy(rows_a, acc.at[pl.ds(base + k * CHUNK, CHUNK)],
                              sem_s)
        for k in range(ROWS_PER_TILE // CHUNK)
    ]
    for zc in zcopies:
        zc.start()
    for zc in zcopies:
        zc.wait()
    plsc.subcore_barrier()

    HALF = CHUNK // 2

    def gather_descs(t, buf):
        return (
            pltpu.make_async_copy(h_hbm.at[src_v.at[t, pl.ds(0, HALF)]],
                                  buf.at[pl.ds(0, HALF)], sem_g),
            pltpu.make_async_copy(h_hbm.at[src_v.at[t, pl.ds(HALF, HALF)]],
                                  buf.at[pl.ds(HALF, HALF)], sem_g),
        )

    def gather_start(t, buf):
        for d in gather_descs(t, buf):
            d.start()

    def gather_wait(t, buf):
        for d in gather_descs(t, buf):
            d.wait()

    def scatter(t, buf):
        return pltpu.async_copy(buf, acc.at[dst_v.at[t]], sem_s, add=True)

    def scatter_wait(t, buf):
        pltpu.make_async_copy(buf, acc.at[dst_v.at[t]], sem_s).wait()

    def super_body(s, _):
        icopies = [
            pltpu.make_async_copy(src_hbm.at[cid, sid, s], src_v, sem_g),
            pltpu.make_async_copy(dst_hbm.at[cid, sid, s], dst_v, sem_g),
            pltpu.make_async_copy(w_hbm.at[cid, sid, s], w_v, sem_g),
        ]
        for ic in icopies:
            ic.start()
        for ic in icopies:
            ic.wait()

        gather_start(0, rows_a)

        def chunk_body(t, _):
            def stage(cur, nxt):
                gather_wait(t, cur)

                @pl.when(t < SUB - 1)
                def _prefetch():
                    @pl.when(t >= 1)
                    def _free():
                        scatter_wait(t - 1, nxt)
                    gather_start(t + 1, nxt)

                _scale_rows(cur, w_v, t)
                scatter(t, cur)

            @pl.when(t % 2 == 0)
            def _even():
                stage(rows_a, rows_b)

            @pl.when(t % 2 == 1)
            def _odd():
                stage(rows_b, rows_a)
            return 0
        lax.fori_loop(0, SUB, chunk_body, 0)

        scatter_wait(SUB - 2, rows_b if (SUB - 2) % 2 else rows_a)
        scatter_wait(SUB - 1, rows_b if (SUB - 1) % 2 else rows_a)
        return 0
    lax.fori_loop(0, NCHUNK // SUB, super_body, 0)
    plsc.subcore_barrier()

    pltpu.sync_copy(acc.at[pl.ds(base, ROWS_PER_TILE)],
                    out_hbm.at[cid, pl.ds(base, ROWS_PER_TILE)])


def _sc_aggregate(h, src_r, dst_r, w_r):
    mesh = plsc.VectorSubcoreMesh(core_axis_name="c", subcore_axis_name="s",
                                  num_cores=NC, num_subcores=NS)
    return pl.kernel(
        _sc_agg_kernel,
        out_type=jax.ShapeDtypeStruct((NC, ACC_N, D), jnp.float32),
        mesh=mesh,
        scratch_types=[
            pltpu.VMEM((SUB, CHUNK), jnp.int32),
            pltpu.VMEM((SUB, CHUNK), jnp.int32),
            pltpu.VMEM((SUB, CHUNK), jnp.float32),
            pltpu.VMEM((CHUNK, D), jnp.float32),
            pltpu.VMEM((CHUNK, D), jnp.float32),
            pltpu.VMEM_SHARED((ACC_N, D), jnp.float32),
            pltpu.SemaphoreType.DMA,
            pltpu.SemaphoreType.DMA,
        ],
    )(h, src_r, dst_r, w_r)


def kernel(x, edge_index, edge_weight, W1, b1, W2, b2):
    eshape = (NC, NS, NCHUNK // SUB, SUB, CHUNK)
    src = edge_index[0].astype(jnp.int32).reshape(eshape)
    dst = edge_index[1].astype(jnp.int32).reshape(eshape)
    w = edge_weight.reshape(eshape)

    h1 = _tc_matmul(x, W1)
    p1 = _sc_aggregate(h1, src, dst, w)
    h2 = _tc_fused_matmul(p1, b1, W2)
    p2 = _sc_aggregate(h2, src, dst, w)
    return _tc_add_bias(p2, b2)


# speedup vs baseline: 8.9753x; 1.0102x over previous
import functools

import jax
import jax.numpy as jnp
from jax import lax
from jax.experimental import pallas as pl
from jax.experimental.pallas import tpu as pltpu
from jax.experimental.pallas import tpu_sc as plsc

N = 10000
E = 320000
D = 128

NC = 2
NS = 16
NW = NC * NS

CHUNK = 80
NCHUNK = E // NW // CHUNK
SUB = 5
NSUPERS = NCHUNK // SUB
ACC_N = 10240
ROWS_PER_TILE = ACC_N // NS


def _mm_body(x_ref, w_ref, o_ref):
    o_ref[...] = jnp.dot(x_ref[...], w_ref[...],
                         preferred_element_type=jnp.float32)


def _tc_matmul(x, w, block=2000):
    grid = (x.shape[0] // block,)
    return pl.pallas_call(
        _mm_body,
        grid=grid,
        in_specs=[
            pl.BlockSpec((block, x.shape[1]), lambda i: (i, 0)),
            pl.BlockSpec((w.shape[0], w.shape[1]), lambda i: (0, 0)),
        ],
        out_specs=pl.BlockSpec((block, w.shape[1]), lambda i: (i, 0)),
        out_shape=jax.ShapeDtypeStruct((x.shape[0], w.shape[1]), jnp.float32),
    )(x, w)


def _fused_mm_body(p_ref, b_ref, w_ref, o_ref):
    h = jax.nn.relu(p_ref[0] + p_ref[1] + b_ref[...])
    o_ref[...] = jnp.dot(h, w_ref[...], preferred_element_type=jnp.float32)


def _tc_fused_matmul(parts, b, w, block=2000):
    grid = (N // block,)
    return pl.pallas_call(
        _fused_mm_body,
        grid=grid,
        in_specs=[
            pl.BlockSpec((2, block, D), lambda i: (0, i, 0)),
            pl.BlockSpec((1, D), lambda i: (0, 0)),
            pl.BlockSpec((D, D), lambda i: (0, 0)),
        ],
        out_specs=pl.BlockSpec((block, D), lambda i: (i, 0)),
        out_shape=jax.ShapeDtypeStruct((N, D), jnp.float32),
    )(parts, b.reshape(1, D), w)


def _bias_body(p_ref, b_ref, o_ref):
    o_ref[...] = p_ref[0] + p_ref[1] + b_ref[...]


def _tc_add_bias(parts, b, block=2000):
    grid = (N // block,)
    return pl.pallas_call(
        _bias_body,
        grid=grid,
        in_specs=[
            pl.BlockSpec((2, block, D), lambda i: (0, i, 0)),
            pl.BlockSpec((1, D), lambda i: (0, 0)),
        ],
        out_specs=pl.BlockSpec((block, D), lambda i: (i, 0)),
        out_shape=jax.ShapeDtypeStruct((N, D), jnp.float32),
    )(parts, b.reshape(1, D))


def _scale_rows(rows_v, w_v, t):
    def grp_body(g, _):
        wvec = w_v[t, pl.ds(g * 16, 16)]
        for l in range(16):
            wl = wvec[l]
            r = g * 16 + l
            for j in range(D // 16):
                sl = pl.ds(j * 16, 16)
                rows_v[r, sl] = rows_v[r, sl] * wl
        return 0
    lax.fori_loop(0, CHUNK // 16, grp_body, 0)


def _sc_agg_kernel(h_hbm, src_hbm, dst_hbm, w_hbm, out_hbm,
                   src0, src1, dst0, dst1, w0, w1, rows_a, rows_b, acc,
                   sem_g, sem_s, sem_i):
    cid = lax.axis_index("c")
    sid = lax.axis_index("s")
    srcs, dsts, ws = (src0, src1), (dst0, dst1), (w0, w1)
    bufs = (rows_a, rows_b)

    def zrow(i, _):
        for j in range(D // 16):
            rows_a[i, pl.ds(j * 16, 16)] = jnp.zeros((16,), jnp.float32)
        return 0
    lax.fori_loop(0, CHUNK, zrow, 0)
    base = sid * ROWS_PER_TILE
    zcopies = [
        pltpu.make_async_copy(rows_a, acc.at[pl.ds(base + k * CHUNK, CHUNK)],
                              sem_s)
        for k in range(ROWS_PER_TILE // CHUNK)
    ]
    for zc in zcopies:
        zc.start()
    for zc in zcopies:
        zc.wait()
    plsc.subcore_barrier()

    HALF = CHUNK // 2

    def gather_start(idx_ref, row, buf):
        pltpu.make_async_copy(h_hbm.at[idx_ref.at[row, pl.ds(0, HALF)]],
                              buf.at[pl.ds(0, HALF)], sem_g).start()
        pltpu.make_async_copy(h_hbm.at[idx_ref.at[row, pl.ds(HALF, HALF)]],
                              buf.at[pl.ds(HALF, HALF)], sem_g).start()

    def gather_wait(buf):
        pltpu.make_async_copy(h_hbm.at[src0.at[0]], buf, sem_g).wait()

    def scatter(idx_ref, row, buf):
        pltpu.async_copy(buf, acc.at[idx_ref.at[row]], sem_s, add=True)

    def scatter_wait(buf):
        pltpu.make_async_copy(buf, acc.at[dst0.at[0]], sem_s).wait()

    def idx_start(s_next, slot):
        pltpu.make_async_copy(src_hbm.at[cid, sid, s_next], srcs[slot],
                              sem_i).start()
        pltpu.make_async_copy(dst_hbm.at[cid, sid, s_next], dsts[slot],
                              sem_i).start()
        pltpu.make_async_copy(w_hbm.at[cid, sid, s_next], ws[slot],
                              sem_i).start()

    def idx_wait(slot):
        pltpu.make_async_copy(src_hbm.at[cid, sid, 0], srcs[slot],
                              sem_i).wait()
        pltpu.make_async_copy(dst_hbm.at[cid, sid, 0], dsts[slot],
                              sem_i).wait()
        pltpu.make_async_copy(w_hbm.at[cid, sid, 0], ws[slot],
                              sem_i).wait()

    idx_start(0, 0)
    idx_wait(0)
    gather_start(src0, 0, rows_a)

    def super_body(s, _):
        def run(par):
            cs, cd, cw = srcs[par], dsts[par], ws[par]
            for b in range(SUB):
                buf = bufs[(par + b) % 2]
                nbuf = bufs[(par + b + 1) % 2]
                t = s * SUB + b
                if b == 0:
                    @pl.when(s < NSUPERS - 1)
                    def _stage_next():
                        idx_start(s + 1, 1 - par)
                gather_wait(buf)
                if b < SUB - 1:
                    @pl.when(t >= 1)
                    def _free():
                        scatter_wait(nbuf)
                    gather_start(cs, b + 1, nbuf)
                else:
                    @pl.when(s < NSUPERS - 1)
                    def _cross():
                        scatter_wait(nbuf)
                        idx_wait(1 - par)
                        gather_start(srcs[1 - par], 0, nbuf)
                _scale_rows(buf, cw, b)
                scatter(cd, b, buf)

        @pl.when(s % 2 == 0)
        def _even():
            run(0)

        @pl.when(s % 2 == 1)
        def _odd():
            run(1)
        return 0
    lax.fori_loop(0, NSUPERS, super_body, 0)

    scatter_wait(rows_a)
    scatter_wait(rows_b)
    plsc.subcore_barrier()

    pltpu.sync_copy(acc.at[pl.ds(base, ROWS_PER_TILE)],
                    out_hbm.at[cid, pl.ds(base, ROWS_PER_TILE)])


def _sc_aggregate(h, src_r, dst_r, w_r):
    mesh = plsc.VectorSubcoreMesh(core_axis_name="c", subcore_axis_name="s",
                                  num_cores=NC, num_subcores=NS)
    return pl.kernel(
        _sc_agg_kernel,
        out_type=jax.ShapeDtypeStruct((NC, ACC_N, D), jnp.float32),
        mesh=mesh,
        scratch_types=[
            pltpu.VMEM((SUB, CHUNK), jnp.int32),
            pltpu.VMEM((SUB, CHUNK), jnp.int32),
            pltpu.VMEM((SUB, CHUNK), jnp.int32),
            pltpu.VMEM((SUB, CHUNK), jnp.int32),
            pltpu.VMEM((SUB, CHUNK), jnp.float32),
            pltpu.VMEM((SUB, CHUNK), jnp.float32),
            pltpu.VMEM((CHUNK, D), jnp.float32),
            pltpu.VMEM((CHUNK, D), jnp.float32),
            pltpu.VMEM_SHARED((ACC_N, D), jnp.float32),
            pltpu.SemaphoreType.DMA,
            pltpu.SemaphoreType.DMA,
            pltpu.SemaphoreType.DMA,
        ],
    )(h, src_r, dst_r, w_r)


def kernel(x, edge_index, edge_weight, W1, b1, W2, b2):
    eshape = (NC, NS, NCHUNK // SUB, SUB, CHUNK)
    src = edge_index[0].astype(jnp.int32).reshape(eshape)
    dst = edge_index[1].astype(jnp.int32).reshape(eshape)
    w = edge_weight.reshape(eshape)

    h1 = _tc_matmul(x, W1)
    p1 = _sc_aggregate(h1, src, dst, w)
    h2 = _tc_fused_matmul(p1, b1, W2)
    p2 = _sc_aggregate(h2, src, dst, w)
    return _tc_add_bias(p2, b2)
